# Initial kernel scaffold; baseline (speedup 1.0000x reference)
import jax
import jax.numpy as jnp
from jax.experimental import pallas as pl


def kernel(input_tokens, W, b):
    raise NotImplementedError("write your pallas kernel here")


# speedup vs baseline: 2.5283x; 2.5283x over previous
import functools

import jax
import jax.numpy as jnp
from jax.experimental import pallas as pl
from jax.experimental.pallas import tpu as pltpu

B, N, DIM, E = 4, 4096, 2048, 8
CAP = 512
TOK_BLK = 512
N_TOK = B * N


def _router_kernel(x_ref, w_ref, b_ref, pt_ref):
    logits = jax.lax.dot_general(
        w_ref[...], x_ref[...],
        dimension_numbers=(((1,), (1,)), ((), ())),
        preferred_element_type=jnp.float32,
    )
    logits = logits + b_ref[...].reshape(E, 1)
    m = jnp.max(logits, axis=0, keepdims=True)
    e = jnp.exp(logits - m)
    pt_ref[...] = e / jnp.sum(e, axis=0, keepdims=True)


def _route_kernel(pt_ref, mask_ref, ep_ref):
    idx = jax.lax.broadcasted_iota(jnp.int32, (1, N), 1)
    unassigned = jnp.ones((1, N), dtype=jnp.int32)
    maskv = jnp.full((1, N), -1, dtype=jnp.int32)

    for j in reversed(range(E)):
        p_j = pt_ref[j]
        keys = jnp.where(unassigned > 0,
                         jax.lax.bitcast_convert_type(p_j, jnp.int32),
                         jnp.int32(-1))

        def val_step(_, lohi):
            lo, hi = lohi
            mid = lo + ((hi - lo + 1) >> 1)
            cnt = jnp.sum((keys >= mid).astype(jnp.int32), axis=1,
                          keepdims=True)
            ok = cnt >= CAP
            return jnp.where(ok, mid, lo), jnp.where(ok, hi, mid - 1)

        lo0 = jnp.full((B, 1), -1, dtype=jnp.int32)
        hi0 = jnp.full((B, 1), 0x3F800000, dtype=jnp.int32)
        K, _ = jax.lax.fori_loop(0, 31, val_step, (lo0, hi0))

        cg = jnp.sum((keys > K).astype(jnp.int32), axis=1, keepdims=True)
        need = CAP - cg

        def idx_step(_, lohi):
            lo, hi = lohi
            mid = (lo + hi) >> 1
            g = jnp.sum(((keys == K) & (idx < mid)).astype(jnp.int32),
                        axis=1, keepdims=True)
            ok = g >= need
            return jnp.where(ok, lo, mid + 1), jnp.where(ok, mid, hi)

        lo0 = jnp.zeros((B, 1), dtype=jnp.int32)
        hi0 = jnp.full((B, 1), N, dtype=jnp.int32)
        C, _ = jax.lax.fori_loop(0, 13, idx_step, (lo0, hi0))

        sel = (keys > K) | ((keys == K) & (idx < C))
        s_union = jnp.max(sel.astype(jnp.int32), axis=0, keepdims=True)
        maskv = jnp.where(s_union > 0, jnp.int32(j), maskv)
        unassigned = unassigned * (1 - s_union)

    maskv = jnp.where(maskv == -1, 0, maskv)
    mask_ref[...] = jnp.broadcast_to(maskv, (B, N))
    ep = jnp.zeros((B, N), dtype=jnp.float32)
    for j in range(E):
        ep = ep + jnp.where(maskv == j, pt_ref[j], 0.0)
    ep_ref[...] = ep


@jax.jit
def kernel(input_tokens, W, b):
    x = input_tokens.reshape(N_TOK, DIM)
    b2 = b.reshape(1, E)
    probs_t = pl.pallas_call(
        _router_kernel,
        grid=(N_TOK // TOK_BLK,),
        in_specs=[
            pl.BlockSpec((TOK_BLK, DIM), lambda i: (i, 0)),
            pl.BlockSpec((E, DIM), lambda i: (0, 0)),
            pl.BlockSpec((1, E), lambda i: (0, 0)),
        ],
        out_specs=pl.BlockSpec((E, TOK_BLK), lambda i: (0, i)),
        out_shape=jax.ShapeDtypeStruct((E, N_TOK), jnp.float32),
    )(x, W, b2)
    probs_t = probs_t.reshape(E, B, N)
    mask, ep = pl.pallas_call(
        _route_kernel,
        out_shape=(
            jax.ShapeDtypeStruct((B, N), jnp.int32),
            jax.ShapeDtypeStruct((B, N), jnp.float32),
        ),
    )(probs_t)
    return mask, ep


# speedup vs baseline: 3.2874x; 1.3002x over previous
import jax
import jax.numpy as jnp
from jax.experimental import pallas as pl

B, N, DIM, E = 4, 4096, 2048, 8
CAP = 512
TOK_BLK = 512
N_TOK = B * N
SUB = 8
LANE = N // SUB


def _router_kernel(x_ref, w_ref, b_ref, pt_ref):
    logits = jax.lax.dot_general(
        w_ref[...], x_ref[...],
        dimension_numbers=(((1,), (1,)), ((), ())),
        preferred_element_type=jnp.float32,
    )
    logits = logits + b_ref[...].reshape(E, 1)
    m = jnp.max(logits, axis=0, keepdims=True)
    e = jnp.exp(logits - m)
    pt_ref[...] = e / jnp.sum(e, axis=0, keepdims=True)


def _count_ge(keys, thr):
    return jnp.sum((keys >= thr).astype(jnp.int32), axis=(1, 2), keepdims=True)


def _route_kernel(pt_ref, mask_ref, ep_ref):
    idx = (LANE * jax.lax.broadcasted_iota(jnp.int32, (1, SUB, LANE), 1)
           + jax.lax.broadcasted_iota(jnp.int32, (1, SUB, LANE), 2))
    unassigned = jnp.ones((1, SUB, LANE), dtype=jnp.int32)
    maskv = jnp.full((1, SUB, LANE), -1, dtype=jnp.int32)

    for j in reversed(range(E)):
        p_j = pt_ref[j]
        keys = jnp.where(unassigned > 0,
                         jax.lax.bitcast_convert_type(p_j, jnp.int32) + 1,
                         jnp.int32(0))

        kth = jnp.zeros((B, 1, 1), dtype=jnp.int32)
        for r in range(10):
            sh = 27 - 3 * r
            cnts = [_count_ge(keys, kth + (d << sh)) for d in range(1, 8)]
            dwin = sum((c >= CAP).astype(jnp.int32) for c in cnts)
            kth = kth + (dwin << sh)

        cg = _count_ge(keys, kth + 1)
        need = CAP - cg
        ties = keys == kth

        def f_lt(m):
            return jnp.sum((ties & (idx < m)).astype(jnp.int32),
                           axis=(1, 2), keepdims=True)

        mcut = jnp.zeros((B, 1, 1), dtype=jnp.int32)
        for r in range(4):
            sh = 9 - 3 * r
            fs = [f_lt(mcut + (d << sh)) for d in range(1, 8)]
            dwin = sum((f < need).astype(jnp.int32) for f in fs)
            mcut = mcut + (dwin << sh)

        sel = (keys > kth) | (ties & (idx <= mcut))
        s_union = jnp.max(sel.astype(jnp.int32), axis=0, keepdims=True)
        maskv = jnp.where(s_union > 0, jnp.int32(j), maskv)
        unassigned = unassigned * (1 - s_union)

    maskv = jnp.where(maskv == -1, 0, maskv)
    mask_ref[...] = jnp.broadcast_to(maskv, (B, SUB, LANE))
    ep = jnp.zeros((B, SUB, LANE), dtype=jnp.float32)
    for j in range(E):
        ep = ep + jnp.where(maskv == j, pt_ref[j], 0.0)
    ep_ref[...] = ep


@jax.jit
def kernel(input_tokens, W, b):
    x = input_tokens.reshape(N_TOK, DIM)
    b2 = b.reshape(1, E)
    probs_t = pl.pallas_call(
        _router_kernel,
        grid=(N_TOK // TOK_BLK,),
        in_specs=[
            pl.BlockSpec((TOK_BLK, DIM), lambda i: (i, 0)),
            pl.BlockSpec((E, DIM), lambda i: (0, 0)),
            pl.BlockSpec((1, E), lambda i: (0, 0)),
        ],
        out_specs=pl.BlockSpec((E, TOK_BLK), lambda i: (0, i)),
        out_shape=jax.ShapeDtypeStruct((E, N_TOK), jnp.float32),
    )(x, W, b2)
    probs_t = probs_t.reshape(E, B, SUB, LANE)
    mask, ep = pl.pallas_call(
        _route_kernel,
        out_shape=(
            jax.ShapeDtypeStruct((B, SUB, LANE), jnp.int32),
            jax.ShapeDtypeStruct((B, SUB, LANE), jnp.float32),
        ),
    )(probs_t)
    return mask.reshape(B, N), ep.reshape(B, N)


# speedup vs baseline: 4.2389x; 1.2895x over previous
import jax
import jax.numpy as jnp
from jax.experimental import pallas as pl
from jax.experimental.pallas import tpu as pltpu

B, N, DIM, E = 4, 4096, 2048, 8
CAP = 512
TOK_BLK = 1024
N_TOK = B * N
SUB = 8
LANE = N // SUB


def _router_kernel(x_ref, w_ref, b_ref, pt_ref):
    logits = jax.lax.dot_general(
        w_ref[...], x_ref[...],
        dimension_numbers=(((1,), (1,)), ((), ())),
        preferred_element_type=jnp.float32,
    )
    logits = logits + b_ref[...].reshape(E, 1)
    m = jnp.max(logits, axis=0, keepdims=True)
    e = jnp.exp(logits - m)
    pt_ref[...] = e / jnp.sum(e, axis=0, keepdims=True)


def _count_ge(keys, thr):
    return jnp.sum((keys >= thr).astype(jnp.float32), axis=(1, 2),
                   keepdims=True)


def _route_kernel(pt_ref, mask_ref, ep_ref, sun_ref):
    idx = (LANE * jax.lax.broadcasted_iota(jnp.int32, (1, SUB, LANE), 1)
           + jax.lax.broadcasted_iota(jnp.int32, (1, SUB, LANE), 2))
    unassigned = jnp.ones((1, SUB, LANE), dtype=jnp.int32)
    maskv = jnp.full((1, SUB, LANE), -1, dtype=jnp.int32)

    for j in reversed(range(E)):
        p_j = pt_ref[j]

        def _search(unassigned=unassigned, p_j=p_j):
            keys = jnp.where(unassigned > 0,
                             jax.lax.bitcast_convert_type(p_j, jnp.int32) + 1,
                             jnp.int32(0))

            kth = jnp.zeros((B, 1, 1), dtype=jnp.int32)
            for r in range(10):
                sh = 27 - 3 * r
                cnts = [_count_ge(keys, kth + (d << sh)) for d in range(1, 8)]
                dwin = sum((c >= float(CAP)).astype(jnp.int32) for c in cnts)
                kth = kth + (dwin << sh)

            cg = _count_ge(keys, kth + 1)
            need = float(CAP) - cg
            ties = keys == kth

            def f_lt(m):
                return jnp.sum((ties & (idx < m)).astype(jnp.float32),
                               axis=(1, 2), keepdims=True)

            mcut = jnp.zeros((B, 1, 1), dtype=jnp.int32)
            for r in range(4):
                sh = 9 - 3 * r
                fs = [f_lt(mcut + (d << sh)) for d in range(1, 8)]
                dwin = sum((f < need).astype(jnp.int32) for f in fs)
                mcut = mcut + (dwin << sh)

            sel = (keys > kth) | (ties & (idx <= mcut))
            return jnp.max(sel.astype(jnp.int32), axis=0, keepdims=True)

        any_unassigned = jnp.sum(unassigned) > 0

        @pl.when(any_unassigned)
        def _():
            sun_ref[...] = _search()

        @pl.when(jnp.logical_not(any_unassigned))
        def _():
            sun_ref[...] = (idx < CAP).astype(jnp.int32)

        s_union = sun_ref[...]
        maskv = jnp.where(s_union > 0, jnp.int32(j), maskv)
        unassigned = unassigned * (1 - s_union)

    maskv = jnp.where(maskv == -1, 0, maskv)
    mask_ref[...] = jnp.broadcast_to(maskv, (B, SUB, LANE))
    ep = jnp.zeros((B, SUB, LANE), dtype=jnp.float32)
    for j in range(E):
        ep = ep + jnp.where(maskv == j, pt_ref[j], 0.0)
    ep_ref[...] = ep


@jax.jit
def kernel(input_tokens, W, b):
    x = input_tokens.reshape(N_TOK, DIM)
    b2 = b.reshape(1, E)
    probs_t = pl.pallas_call(
        _router_kernel,
        grid=(N_TOK // TOK_BLK,),
        in_specs=[
            pl.BlockSpec((TOK_BLK, DIM), lambda i: (i, 0)),
            pl.BlockSpec((E, DIM), lambda i: (0, 0)),
            pl.BlockSpec((1, E), lambda i: (0, 0)),
        ],
        out_specs=pl.BlockSpec((E, TOK_BLK), lambda i: (0, i)),
        out_shape=jax.ShapeDtypeStruct((E, N_TOK), jnp.float32),
    )(x, W, b2)
    probs_t = probs_t.reshape(E, B, SUB, LANE)
    mask, ep = pl.pallas_call(
        _route_kernel,
        out_shape=(
            jax.ShapeDtypeStruct((B, SUB, LANE), jnp.int32),
            jax.ShapeDtypeStruct((B, SUB, LANE), jnp.float32),
        ),
        scratch_shapes=[pltpu.VMEM((1, SUB, LANE), jnp.int32)],
    )(probs_t)
    return mask.reshape(B, N), ep.reshape(B, N)
